# baseline (speedup 1.0000x reference)
import jax
import jax.numpy as jnp
from jax import lax
from jax.experimental import pallas as pl
from jax.experimental.pallas import tpu as pltpu
from jax.experimental.pallas import tpu_sc as plsc

BASE_RATIO = 0.25
FINAL_RATIO = 0.5
EPOCHS_TOTAL = 100

B0, N0, C0, L = 8, 16, 32, 8192
R = B0 * N0 * C0
NLANE = 16
NCH = L // NLANE
NCORES = 2
NSUB = 16
NW = NCORES * NSUB
RPW = R // NW

MASK31 = 0x7FFFFFFF


def _suffix(v):
    incl = lax.rev(plsc.cumsum(lax.rev(v, (0,))), (0,))
    return incl, incl - v


def _scan_stage(fine_ref, k, iota16):
    tvec = plsc.load_gather(fine_ref, [iota16 * NLANE])
    for j in range(1, NLANE):
        tvec = tvec + plsc.load_gather(fine_ref, [iota16 * NLANE + j])
    incl, excl = _suffix(tvec)
    hit = (excl < k) & (incl >= k)
    c1 = jnp.max(jnp.where(hit, excl * NLANE + iota16, -1))
    istar = c1 & (NLANE - 1)
    k2 = k - (c1 >> 4)
    v = fine_ref[pl.ds(istar * NLANE, NLANE)]
    incl2, excl2 = _suffix(v)
    hit2 = (excl2 < k2) & (incl2 >= k2)
    c2 = jnp.max(jnp.where(hit2, excl2 * NLANE + iota16, -1))
    return istar * NLANE + (c2 & (NLANE - 1)), k2 - (c2 >> 4)


def _sc_body(x_hbm, k_hbm, out_hbm,
             xin0, xin1, mout0, mout1, kvm,
             fine_a, fine_b, fine_c, fine_d,
             insem0, insem1, outsem0, outsem1):
    wid = lax.axis_index("s") * NCORES + lax.axis_index("c")
    base = wid * RPW
    pltpu.sync_copy(k_hbm, kvm)
    k0 = jnp.max(kvm[...])
    k0 = jnp.minimum(jnp.maximum(k0, jnp.int32(1)), jnp.int32(L))
    iota16 = lax.iota(jnp.int32, NLANE)
    ones16 = jnp.ones((NLANE,), jnp.int32)
    zeros16 = jnp.zeros((NLANE,), jnp.int32)
    insems = (insem0, insem1)
    outsems = (outsem0, outsem1)
    xins = (xin0, xin1)
    mouts = (mout0, mout1)

    pltpu.async_copy(x_hbm.at[base], xin0, insem0)

    def row_pair(r2, carry):
        for s in range(2):
            r = r2 * 2 + s

            @pl.when(r + 1 < RPW)
            def _start_next():
                pltpu.async_copy(
                    x_hbm.at[base + r + 1], xins[1 - s], insems[1 - s])

            pltpu.make_async_copy(
                x_hbm.at[base + r], xins[s], insems[s]).wait()

            @pl.when(r >= 2)
            def _drain_out():
                pltpu.make_async_copy(
                    mouts[s], out_hbm.at[base + r - 2], outsems[s]).wait()

            xrow = xins[s]
            mrow = mouts[s]

            for i in range(16):
                fine_a[pl.ds(i * NLANE, NLANE)] = zeros16
                fine_b[pl.ds(i * NLANE, NLANE)] = zeros16
                fine_c[pl.ds(i * NLANE, NLANE)] = zeros16
                fine_d[pl.ds(i * NLANE, NLANE)] = zeros16

            @plsc.parallel_loop(0, NCH, unroll=8)
            def p1(c):
                v = xrow[pl.ds(c * NLANE, NLANE)]
                b = plsc.bitcast(v, jnp.int32) & MASK31
                e = b >> 23
                occ, last = plsc.scan_count(e)
                plsc.addupdate_scatter(fine_a, [e], occ, mask=last)

            estar, k1 = _scan_stage(fine_a, k0, iota16)

            @plsc.parallel_loop(0, NCH, unroll=8)
            def p2(c):
                v = xrow[pl.ds(c * NLANE, NLANE)]
                b = plsc.bitcast(v, jnp.int32) & MASK31
                bkt = (b >> 15) - (estar << 8)
                m = plsc.bitcast(bkt, jnp.uint32) < 256
                plsc.addupdate_scatter(fine_b, [bkt], ones16, mask=m)

            b2star, k2 = _scan_stage(fine_b, k1, iota16)
            p16 = (estar << 8) | b2star

            @plsc.parallel_loop(0, NCH, unroll=8)
            def p3(c):
                v = xrow[pl.ds(c * NLANE, NLANE)]
                b = plsc.bitcast(v, jnp.int32) & MASK31
                bkt = (b >> 7) - (p16 << 8)
                m = plsc.bitcast(bkt, jnp.uint32) < 256
                plsc.addupdate_scatter(fine_c, [bkt], ones16, mask=m)

            b3star, k3 = _scan_stage(fine_c, k2, iota16)
            p24 = (p16 << 8) | b3star

            @plsc.parallel_loop(0, NCH, unroll=8)
            def p4(c):
                v = xrow[pl.ds(c * NLANE, NLANE)]
                b = plsc.bitcast(v, jnp.int32) & MASK31
                bkt = b - (p24 << 7)
                m = plsc.bitcast(bkt, jnp.uint32) < 128
                plsc.addupdate_scatter(fine_d, [bkt], ones16, mask=m)

            b4star, _k4 = _scan_stage(fine_d, k3, iota16)
            tbits = (p24 << 7) | b4star

            @plsc.parallel_loop(0, NCH, unroll=8)
            def pm(c):
                v = xrow[pl.ds(c * NLANE, NLANE)]
                b = plsc.bitcast(v, jnp.int32) & MASK31
                mrow[pl.ds(c * NLANE, NLANE)] = jnp.where(
                    b >= tbits, jnp.float32(1.0), jnp.float32(0.0))

            pltpu.async_copy(mrow, out_hbm.at[base + r], outsems[s])
        return carry

    lax.fori_loop(0, RPW // 2, row_pair, 0)
    pltpu.make_async_copy(
        mout0, out_hbm.at[base + RPW - 2], outsems[0]).wait()
    pltpu.make_async_copy(
        mout1, out_hbm.at[base + RPW - 1], outsems[1]).wait()


_sc_kernel = pl.kernel(
    _sc_body,
    out_type=jax.ShapeDtypeStruct((R, L), jnp.float32),
    mesh=plsc.VectorSubcoreMesh(core_axis_name="c", subcore_axis_name="s"),
    scratch_types=[
        pltpu.VMEM((L,), jnp.float32),
        pltpu.VMEM((L,), jnp.float32),
        pltpu.VMEM((L,), jnp.float32),
        pltpu.VMEM((L,), jnp.float32),
        pltpu.VMEM((NLANE,), jnp.int32),
        pltpu.VMEM((256,), jnp.int32),
        pltpu.VMEM((256,), jnp.int32),
        pltpu.VMEM((256,), jnp.int32),
        pltpu.VMEM((256,), jnp.int32),
        pltpu.SemaphoreType.DMA,
        pltpu.SemaphoreType.DMA,
        pltpu.SemaphoreType.DMA,
        pltpu.SemaphoreType.DMA,
    ],
    compiler_params=pltpu.CompilerParams(needs_layout_passes=False),
)


def kernel(x, epoch):
    ratio = BASE_RATIO + (FINAL_RATIO - BASE_RATIO) * jnp.minimum(
        1.0, epoch / (EPOCHS_TOTAL * 0.8))
    num_mask = jnp.minimum(jnp.floor(L * ratio).astype(jnp.int32), L)
    karr = jnp.full((NLANE,), 1, jnp.int32) * num_mask
    out = _sc_kernel(x.reshape(R, L), karr)
    return out.reshape(x.shape)


# speedup vs baseline: 1.0405x; 1.0405x over previous
import jax
import jax.numpy as jnp
from jax import lax
from jax.experimental import pallas as pl
from jax.experimental.pallas import tpu as pltpu
from jax.experimental.pallas import tpu_sc as plsc

BASE_RATIO = 0.25
FINAL_RATIO = 0.5
EPOCHS_TOTAL = 100

B0, N0, C0, L = 8, 16, 32, 8192
R = B0 * N0 * C0
NLANE = 16
NCH = L // NLANE
NCORES = 2
NSUB = 16
NW = NCORES * NSUB
RPW = R // NW

MASK31 = 0x7FFFFFFF


def _suffix(v):
    incl = lax.rev(plsc.cumsum(lax.rev(v, (0,))), (0,))
    return incl, incl - v


def _scan_stage(fine_ref, k, iota16):
    tvec = plsc.load_gather(fine_ref, [iota16 * NLANE])
    for j in range(1, NLANE):
        tvec = tvec + plsc.load_gather(fine_ref, [iota16 * NLANE + j])
    incl, excl = _suffix(tvec)
    hit = (excl < k) & (incl >= k)
    c1 = jnp.max(jnp.where(hit, excl * NLANE + iota16, -1))
    istar = c1 & (NLANE - 1)
    k2 = k - (c1 >> 4)
    v = fine_ref[pl.ds(istar * NLANE, NLANE)]
    incl2, excl2 = _suffix(v)
    hit2 = (excl2 < k2) & (incl2 >= k2)
    c2 = jnp.max(jnp.where(hit2, excl2 * NLANE + iota16, -1))
    return istar * NLANE + (c2 & (NLANE - 1)), k2 - (c2 >> 4)


def _sc_body(x_hbm, k_hbm, out_hbm,
             xin0, xin1, mout0, mout1, bits, kvm,
             fine_a, fine_b, fine_c, fine_d,
             insem0, insem1, outsem0, outsem1):
    wid = lax.axis_index("s") * NCORES + lax.axis_index("c")
    base = wid * RPW
    pltpu.sync_copy(k_hbm, kvm)
    k0 = jnp.max(kvm[...])
    k0 = jnp.minimum(jnp.maximum(k0, jnp.int32(1)), jnp.int32(L))
    iota16 = lax.iota(jnp.int32, NLANE)
    ones16 = jnp.ones((NLANE,), jnp.int32)
    zeros16 = jnp.zeros((NLANE,), jnp.int32)
    insems = (insem0, insem1)
    outsems = (outsem0, outsem1)
    xins = (xin0, xin1)
    mouts = (mout0, mout1)

    pltpu.async_copy(x_hbm.at[base], xin0, insem0)

    def row_pair(r2, carry):
        for s in range(2):
            r = r2 * 2 + s

            @pl.when(r + 1 < RPW)
            def _start_next():
                pltpu.async_copy(
                    x_hbm.at[base + r + 1], xins[1 - s], insems[1 - s])

            pltpu.make_async_copy(
                x_hbm.at[base + r], xins[s], insems[s]).wait()

            @pl.when(r >= 2)
            def _drain_out():
                pltpu.make_async_copy(
                    mouts[s], out_hbm.at[base + r - 2], outsems[s]).wait()

            xrow = xins[s]
            mrow = mouts[s]

            for i in range(16):
                fine_a[pl.ds(i * NLANE, NLANE)] = zeros16
                fine_b[pl.ds(i * NLANE, NLANE)] = zeros16
                fine_c[pl.ds(i * NLANE, NLANE)] = zeros16
                fine_d[pl.ds(i * NLANE, NLANE)] = zeros16

            @plsc.parallel_loop(0, NCH, unroll=8)
            def p1(c):
                v = xrow[pl.ds(c * NLANE, NLANE)]
                b = plsc.bitcast(v, jnp.int32) & MASK31
                bits[pl.ds(c * NLANE, NLANE)] = b
                e = b >> 23
                occ, last = plsc.scan_count(e)
                plsc.addupdate_scatter(fine_a, [e], occ, mask=last)

            estar, k1 = _scan_stage(fine_a, k0, iota16)

            @plsc.parallel_loop(0, NCH, unroll=8)
            def p2(c):
                b = bits[pl.ds(c * NLANE, NLANE)]
                bkt = (b >> 15) - (estar << 8)
                m = plsc.bitcast(bkt, jnp.uint32) < 256
                plsc.addupdate_scatter(fine_b, [bkt], ones16, mask=m)

            b2star, k2 = _scan_stage(fine_b, k1, iota16)
            p16 = (estar << 8) | b2star

            @plsc.parallel_loop(0, NCH, unroll=8)
            def p3(c):
                b = bits[pl.ds(c * NLANE, NLANE)]
                bkt = (b >> 7) - (p16 << 8)
                m = plsc.bitcast(bkt, jnp.uint32) < 256
                plsc.addupdate_scatter(fine_c, [bkt], ones16, mask=m)

            b3star, k3 = _scan_stage(fine_c, k2, iota16)
            p24 = (p16 << 8) | b3star

            @plsc.parallel_loop(0, NCH, unroll=8)
            def p4(c):
                b = bits[pl.ds(c * NLANE, NLANE)]
                bkt = b - (p24 << 7)
                m = plsc.bitcast(bkt, jnp.uint32) < 128
                plsc.addupdate_scatter(fine_d, [bkt], ones16, mask=m)

            b4star, _k4 = _scan_stage(fine_d, k3, iota16)
            tbits = (p24 << 7) | b4star

            @plsc.parallel_loop(0, NCH, unroll=8)
            def pm(c):
                b = bits[pl.ds(c * NLANE, NLANE)]
                mrow[pl.ds(c * NLANE, NLANE)] = jnp.where(
                    b >= tbits, jnp.float32(1.0), jnp.float32(0.0))

            pltpu.async_copy(mrow, out_hbm.at[base + r], outsems[s])
        return carry

    lax.fori_loop(0, RPW // 2, row_pair, 0)
    pltpu.make_async_copy(
        mout0, out_hbm.at[base + RPW - 2], outsems[0]).wait()
    pltpu.make_async_copy(
        mout1, out_hbm.at[base + RPW - 1], outsems[1]).wait()


_sc_kernel = pl.kernel(
    _sc_body,
    out_type=jax.ShapeDtypeStruct((R, L), jnp.float32),
    mesh=plsc.VectorSubcoreMesh(core_axis_name="c", subcore_axis_name="s"),
    scratch_types=[
        pltpu.VMEM((L,), jnp.float32),
        pltpu.VMEM((L,), jnp.float32),
        pltpu.VMEM((L,), jnp.float32),
        pltpu.VMEM((L,), jnp.float32),
        pltpu.VMEM((L,), jnp.int32),
        pltpu.VMEM((NLANE,), jnp.int32),
        pltpu.VMEM((256,), jnp.int32),
        pltpu.VMEM((256,), jnp.int32),
        pltpu.VMEM((256,), jnp.int32),
        pltpu.VMEM((256,), jnp.int32),
        pltpu.SemaphoreType.DMA,
        pltpu.SemaphoreType.DMA,
        pltpu.SemaphoreType.DMA,
        pltpu.SemaphoreType.DMA,
    ],
    compiler_params=pltpu.CompilerParams(needs_layout_passes=False),
)


def kernel(x, epoch):
    ratio = BASE_RATIO + (FINAL_RATIO - BASE_RATIO) * jnp.minimum(
        1.0, epoch / (EPOCHS_TOTAL * 0.8))
    num_mask = jnp.minimum(jnp.floor(L * ratio).astype(jnp.int32), L)
    karr = jnp.full((NLANE,), 1, jnp.int32) * num_mask
    out = _sc_kernel(x.reshape(R, L), karr)
    return out.reshape(x.shape)
